# Initial kernel scaffold; baseline (speedup 1.0000x reference)
import functools

import jax
import jax.numpy as jnp
from jax import lax
from jax.experimental import pallas as pl
from jax.experimental.pallas import tpu as pltpu
from jax.experimental.pallas import tpu_sc as plsc

_NC = 2
_NS = 16
_NW = _NC * _NS
_CH = 32


@functools.partial(jax.jit, static_argnames=("n_rows", "d"))
def _sc_gather(idx_flat, table, *, n_rows, d):
    b_per_w = n_rows // _NW
    n_chunks = b_per_w // _CH
    mesh = plsc.VectorSubcoreMesh(core_axis_name="c", subcore_axis_name="s")

    @functools.partial(
        pl.kernel,
        out_type=jax.ShapeDtypeStruct((n_rows, d), jnp.float32),
        mesh=mesh,
        scratch_types=[
            pltpu.VMEM((b_per_w,), jnp.int32),
            pltpu.VMEM((2, _CH, d), jnp.float32),
            pltpu.SemaphoreType.DMA((2,)),
            pltpu.SemaphoreType.DMA((2,)),
        ],
    )
    def k(idx_hbm, table_hbm, out_hbm, idx_v, bufs, gsem, osem):
        wid = lax.axis_index("s") * _NC + lax.axis_index("c")
        base = wid * b_per_w
        pltpu.sync_copy(idx_hbm.at[pl.ds(base, b_per_w)], idx_v)

        pltpu.async_copy(
            table_hbm.at[idx_v.at[pl.ds(0, _CH)]], bufs.at[0], gsem.at[0]
        )
        for c in range(n_chunks):
            b = c % 2
            nb = (c + 1) % 2
            if c + 1 < n_chunks:
                if c >= 1:
                    pltpu.make_async_copy(
                        bufs.at[nb], out_hbm.at[pl.ds(0, _CH)], osem.at[nb]
                    ).wait()
                pltpu.async_copy(
                    table_hbm.at[idx_v.at[pl.ds((c + 1) * _CH, _CH)]],
                    bufs.at[nb],
                    gsem.at[nb],
                )
            pltpu.make_async_copy(
                table_hbm.at[idx_v.at[pl.ds(c * _CH, _CH)]], bufs.at[b], gsem.at[b]
            ).wait()
            pltpu.async_copy(
                bufs.at[b], out_hbm.at[pl.ds(base + c * _CH, _CH)], osem.at[b]
            )
        pltpu.make_async_copy(
            bufs.at[(n_chunks - 1) % 2],
            out_hbm.at[pl.ds(0, _CH)],
            osem.at[(n_chunks - 1) % 2],
        ).wait()
        if n_chunks >= 2:
            pltpu.make_async_copy(
                bufs.at[n_chunks % 2],
                out_hbm.at[pl.ds(0, _CH)],
                osem.at[n_chunks % 2],
            ).wait()

    return k(idx_flat, table)


def kernel(input_ids, shared_weight):
    b, s = input_ids.shape
    v, d = shared_weight.shape
    idx_flat = input_ids.reshape(b * s).astype(jnp.int32)
    out = _sc_gather(idx_flat, shared_weight, n_rows=b * s, d=d)
    return out.reshape(b, s, d)


# speedup vs baseline: 1.6267x; 1.6267x over previous
import functools

import jax
import jax.numpy as jnp
from jax import lax
from jax.experimental import pallas as pl
from jax.experimental.pallas import tpu as pltpu
from jax.experimental.pallas import tpu_sc as plsc

_NC = 2
_NS = 16
_NW = _NC * _NS
_CH = 32


@functools.partial(jax.jit, static_argnames=("n_rows", "d"))
def _sc_gather(idx_flat, table, *, n_rows, d):
    b_per_w = n_rows // _NW
    n_chunks = b_per_w // _CH
    mesh = plsc.VectorSubcoreMesh(core_axis_name="c", subcore_axis_name="s")

    @functools.partial(
        pl.kernel,
        out_type=jax.ShapeDtypeStruct((n_rows, d), jnp.float32),
        mesh=mesh,
        scratch_types=[
            pltpu.VMEM((b_per_w,), jnp.int32),
            pltpu.VMEM((2, _CH, d), jnp.float32),
            pltpu.SemaphoreType.DMA((2,)),
            pltpu.SemaphoreType.DMA((2,)),
        ],
    )
    def k(idx_hbm, table_hbm, out_hbm, idx_v, bufs, gsem, osem):
        wid = lax.axis_index("s") * _NC + lax.axis_index("c")
        base = wid * b_per_w
        pltpu.sync_copy(idx_hbm.at[pl.ds(base, b_per_w)], idx_v)

        def gather(c, b):
            return pltpu.async_copy(
                table_hbm.at[idx_v.at[pl.ds(c * _CH, _CH)]], bufs.at[b], gsem.at[b]
            )

        def put(c, b):
            return pltpu.async_copy(
                bufs.at[b], out_hbm.at[pl.ds(base + c * _CH, _CH)], osem.at[b]
            )

        gdesc = [gather(0, 0), None]
        odesc = [None, None]
        for c in range(n_chunks):
            b = c % 2
            nb = 1 - b
            if c + 1 < n_chunks:
                if odesc[nb] is not None:
                    odesc[nb].wait()
                gdesc[nb] = gather(c + 1, nb)
            gdesc[b].wait()
            odesc[b] = put(c, b)
        odesc[(n_chunks - 1) % 2].wait()
        if n_chunks >= 2:
            odesc[(n_chunks - 2) % 2].wait()

    return k(idx_flat, table)


def kernel(input_ids, shared_weight):
    b, s = input_ids.shape
    v, d = shared_weight.shape
    idx_flat = input_ids.reshape(b * s).astype(jnp.int32)
    out = _sc_gather(idx_flat, shared_weight, n_rows=b * s, d=d)
    return out.reshape(b, s, d)
